# baseline (speedup 1.0000x reference)
import functools
import jax
import jax.numpy as jnp
from jax import lax
from jax.experimental import pallas as pl
from jax.experimental.pallas import tpu as pltpu, tpu_sc as plsc

_N = 10000
_NP = 10240
_E = 320000
_EH = _E // 2
_DN = 128
_DE = 16
_HW = 128
_EIN = _DE + 2 * _DN
_NIN = _DE + _DN
_STW = 16
_SW = 32

_C = 128
_NCHUNK = _E // _C
_HCHUNK = _EH // _C
_NW = 32
_NSUB = 16
_NPAD = 10240
_ROWS_PER_SUB = _NPAD // _NSUB

_BN = 2048
_BNO = 2000
_BE = 3200
_CPB = _BE // _C


def _table_body(v_ref, w1_ref, lnw_ref, t_ref, u_ref, s_ref):
    w = lnw_ref[0, :]
    vb = v_ref[...]
    rs = jnp.sum(vb, axis=1, keepdims=True)
    ss = jnp.sum(vb * vb, axis=1, keepdims=True)
    vr = vb * w[_DE:_DE + _DN][None, :]
    vc = vb * w[_DE + _DN:][None, :]
    t_ref[...] = jnp.dot(vr, w1_ref[_DE:_DE + _DN, :],
                         preferred_element_type=jnp.float32).astype(jnp.bfloat16)
    u_ref[...] = jnp.dot(vc, w1_ref[_DE + _DN:, :],
                         preferred_element_type=jnp.float32).astype(jnp.bfloat16)
    pad = jnp.zeros((vb.shape[0], _STW - 2), jnp.float32)
    s_ref[...] = jnp.concatenate([rs, ss, pad], axis=1)


def _make_tables(vpad, eW1, e_ln_w):
    return pl.pallas_call(
        _table_body,
        grid=(_NP // _BN,),
        in_specs=[
            pl.BlockSpec((_BN, _DN), lambda i: (i, 0)),
            pl.BlockSpec((_EIN, _HW), lambda i: (0, 0)),
            pl.BlockSpec((1, _EIN), lambda i: (0, 0)),
        ],
        out_specs=[
            pl.BlockSpec((_BN, _DN), lambda i: (i, 0)),
            pl.BlockSpec((_BN, _DN), lambda i: (i, 0)),
            pl.BlockSpec((_BN, _STW), lambda i: (i, 0)),
        ],
        out_shape=[
            jax.ShapeDtypeStruct((_NP, _DN), jnp.bfloat16),
            jax.ShapeDtypeStruct((_NP, _DN), jnp.bfloat16),
            jax.ShapeDtypeStruct((_NP, _STW), jnp.float32),
        ],
    )(vpad, eW1, e_ln_w.reshape(1, _EIN))


def _gather_body(off, nchunks, t_hbm, u_hbm, s_hbm, row_hbm, col_hbm,
                 g_hbm, rs_hbm, ss_hbm,
                 idx_r, idx_c, buf, sbuf, rsb, ssb,
                 sem_i, sem_t0, sem_t1, sem_s0, sem_s1, sem_w):
    cid = lax.axis_index("c")
    sid = lax.axis_index("s")
    wid = sid * 2 + cid
    q, r = divmod(nchunks, _NW)
    nc = q + jnp.where(wid < r, 1, 0)
    s0 = wid * q + jnp.minimum(wid, r)
    npairs = nc // 2
    odd = nc - npairs * 2
    iota = lax.iota(jnp.int32, 16)
    zer = jnp.zeros((16,), jnp.int32)
    one = jnp.ones((16,), jnp.int32)

    def extract(half, _):
        def scal(j, _):
            rows = half * _C + j * 16 + iota
            rsb[half, pl.ds(j * 16, 16)] = plsc.load_gather(sbuf, [rows, zer])
            ssb[half, pl.ds(j * 16, 16)] = plsc.load_gather(sbuf, [rows, one])
            return 0
        lax.fori_loop(0, _C // 16, scal, 0)

    def pair(i, _):
        ch = s0 + i * 2
        base = (off + ch) * _C
        gbase = ch * _C

        pltpu.async_copy(row_hbm.at[pl.ds(base, 2 * _C)], idx_r, sem_i)
        pltpu.async_copy(col_hbm.at[pl.ds(base, 2 * _C)], idx_c, sem_i)

        @pl.when(i > 0)
        def _():
            pltpu.make_async_copy(buf, g_hbm.at[pl.ds(gbase, 2 * _C)], sem_w).wait()
            pltpu.make_async_copy(rsb, rs_hbm.at[pl.ds(ch, 2)], sem_w).wait()
            pltpu.make_async_copy(ssb, ss_hbm.at[pl.ds(ch, 2)], sem_w).wait()

        pltpu.make_async_copy(row_hbm.at[pl.ds(base, 2 * _C)], idx_r, sem_i).wait()
        pltpu.make_async_copy(col_hbm.at[pl.ds(base, 2 * _C)], idx_c, sem_i).wait()

        ir0 = idx_r.at[pl.ds(0, _C)]
        ir1 = idx_r.at[pl.ds(_C, _C)]
        ic0 = idx_c.at[pl.ds(0, _C)]
        ic1 = idx_c.at[pl.ds(_C, _C)]
        b0 = buf.at[pl.ds(0, _C)]
        b1 = buf.at[pl.ds(_C, _C)]
        sb0 = sbuf.at[pl.ds(0, _C)]
        sb1 = sbuf.at[pl.ds(_C, _C)]

        d_t0 = pltpu.async_copy(t_hbm.at[ir0], b0, sem_t0)
        d_t1 = pltpu.async_copy(t_hbm.at[ir1], b1, sem_t1)
        d_s0 = pltpu.async_copy(s_hbm.at[ir0], sb0, sem_s0)
        d_s1 = pltpu.async_copy(s_hbm.at[ir1], sb1, sem_s1)
        d_t0.wait()
        d_u0 = pltpu.async_copy(u_hbm.at[ic0], b0, sem_t0, add=True)
        d_s0.wait()
        d_v0 = pltpu.async_copy(s_hbm.at[ic0], sb0, sem_s0, add=True)
        d_t1.wait()
        d_u1 = pltpu.async_copy(u_hbm.at[ic1], b1, sem_t1, add=True)
        d_s1.wait()
        d_v1 = pltpu.async_copy(s_hbm.at[ic1], sb1, sem_s1, add=True)
        d_u0.wait()
        d_v0.wait()
        extract(0, None)
        d_u1.wait()
        d_v1.wait()
        extract(1, None)
        pltpu.async_copy(buf, g_hbm.at[pl.ds(gbase, 2 * _C)], sem_w)
        pltpu.async_copy(rsb, rs_hbm.at[pl.ds(ch, 2)], sem_w)
        pltpu.async_copy(ssb, ss_hbm.at[pl.ds(ch, 2)], sem_w)
        return 0

    lax.fori_loop(0, npairs, pair, 0)

    @pl.when(odd == 1)
    def _():
        ch = s0 + npairs * 2
        base = (off + ch) * _C
        ir0 = idx_r.at[pl.ds(0, _C)]
        ic0 = idx_c.at[pl.ds(0, _C)]
        b0 = buf.at[pl.ds(0, _C)]
        sb0 = sbuf.at[pl.ds(0, _C)]
        pltpu.async_copy(row_hbm.at[pl.ds(base, _C)], ir0, sem_i)
        pltpu.async_copy(col_hbm.at[pl.ds(base, _C)], ic0, sem_i)
        pltpu.make_async_copy(buf, g_hbm.at[pl.ds(0, 2 * _C)], sem_w).wait()
        pltpu.make_async_copy(rsb, rs_hbm.at[pl.ds(0, 2)], sem_w).wait()
        pltpu.make_async_copy(ssb, ss_hbm.at[pl.ds(0, 2)], sem_w).wait()
        pltpu.make_async_copy(row_hbm.at[pl.ds(base, _C)], ir0, sem_i).wait()
        pltpu.make_async_copy(col_hbm.at[pl.ds(base, _C)], ic0, sem_i).wait()
        d_t0 = pltpu.async_copy(t_hbm.at[ir0], b0, sem_t0)
        d_s0 = pltpu.async_copy(s_hbm.at[ir0], sb0, sem_s0)
        d_t0.wait()
        d_u0 = pltpu.async_copy(u_hbm.at[ic0], b0, sem_t0, add=True)
        d_s0.wait()
        d_v0 = pltpu.async_copy(s_hbm.at[ic0], sb0, sem_s0, add=True)
        d_u0.wait()
        d_v0.wait()
        extract(0, None)
        pltpu.sync_copy(b0, g_hbm.at[pl.ds(ch * _C, _C)])
        pltpu.sync_copy(rsb.at[0], rs_hbm.at[ch])
        pltpu.sync_copy(ssb.at[0], ss_hbm.at[ch])

    @pl.when(odd == 0)
    def _():
        pltpu.make_async_copy(buf, g_hbm.at[pl.ds(0, 2 * _C)], sem_w).wait()
        pltpu.make_async_copy(rsb, rs_hbm.at[pl.ds(0, 2)], sem_w).wait()
        pltpu.make_async_copy(ssb, ss_hbm.at[pl.ds(0, 2)], sem_w).wait()


def _gather(tbl, utbl, stbl, row, col, off, nchunks):
    mesh = plsc.VectorSubcoreMesh(core_axis_name="c", subcore_axis_name="s")
    k = functools.partial(
        pl.kernel,
        mesh=mesh,
        out_type=[
            jax.ShapeDtypeStruct((nchunks * _C, _DN), jnp.bfloat16),
            jax.ShapeDtypeStruct((nchunks, _C), jnp.float32),
            jax.ShapeDtypeStruct((nchunks, _C), jnp.float32),
        ],
        scratch_types=[
            pltpu.VMEM((2 * _C,), jnp.int32),
            pltpu.VMEM((2 * _C,), jnp.int32),
            pltpu.VMEM((2 * _C, _DN), jnp.bfloat16),
            pltpu.VMEM((2 * _C, _STW), jnp.float32),
            pltpu.VMEM((2, _C), jnp.float32),
            pltpu.VMEM((2, _C), jnp.float32),
            pltpu.SemaphoreType.DMA,
            pltpu.SemaphoreType.DMA,
            pltpu.SemaphoreType.DMA,
            pltpu.SemaphoreType.DMA,
            pltpu.SemaphoreType.DMA,
            pltpu.SemaphoreType.DMA,
        ],
        compiler_params=pltpu.CompilerParams(use_tc_tiling_on_sc=False,
                                             needs_layout_passes=False),
    )(functools.partial(_gather_body, off, nchunks))
    return k(tbl, utbl, stbl, row, col)


def _colbcast(mat):
    t = mat.T
    return jnp.concatenate(
        [jnp.broadcast_to(t[:, a:a + 1], (_C, _DN)) for a in range(_CPB)], axis=0)


def _edge_body(g_ref, rs_ref, ss_ref, e_ref, w1_ref, w2_ref, el_ref, lnw_ref,
               lnb_ref, b1_ref, b2_ref, eo_ref, sc_ref):
    w = lnw_ref[0, :]
    bvec = lnb_ref[0, :]
    u = jnp.dot(w, w1_ref[...], preferred_element_type=jnp.float32)
    c = jnp.dot(bvec, w1_ref[...], preferred_element_type=jnp.float32)
    eb = e_ref[...]
    rs_e = jnp.sum(eb, axis=1, keepdims=True)
    ss_e = jnp.sum(eb * eb, axis=1, keepdims=True)
    ew = eb * w[:_DE][None, :]
    ze = jnp.dot(ew, w1_ref[:_DE, :], preferred_element_type=jnp.float32)
    m = (rs_e + _colbcast(rs_ref[0])) / float(_EIN)
    q = (ss_e + _colbcast(ss_ref[0])) / float(_EIN)
    var = q - m * m
    inv = lax.rsqrt(var + 1e-5)
    z2 = g_ref[...].astype(jnp.float32) + ze - m * u[None, :]
    h1 = jnp.maximum(z2 * inv + (c + b1_ref[0, :])[None, :], 0.0)
    h2 = jnp.maximum(
        jnp.dot(h1, w2_ref[...], preferred_element_type=jnp.float32) + b2_ref[0, :][None, :],
        0.0)
    eo = jnp.dot(eb, el_ref[...], preferred_element_type=jnp.float32) + h2
    eo_ref[...] = eo
    n = eo.shape[0]
    sc_ref[...] = jnp.concatenate(
        [eo, jnp.ones((n, 1), jnp.float32), jnp.zeros((n, _SW - _DE - 1), jnp.float32)],
        axis=1)


def _edge_mlp(g, rs2d, ss2d, e, eW1, eW2, edge_linear, e_ln_w, e_ln_b, eb1, eb2,
              off_chunks, nchunks):
    off_b = off_chunks // _CPB
    nblk = nchunks // _CPB
    return pl.pallas_call(
        _edge_body,
        grid=(nblk,),
        in_specs=[
            pl.BlockSpec((_BE, _DN), lambda i: (i, 0)),
            pl.BlockSpec((1, _CPB, _C), lambda i: (i, 0, 0)),
            pl.BlockSpec((1, _CPB, _C), lambda i: (i, 0, 0)),
            pl.BlockSpec((_BE, _DE), lambda i: (i + off_b, 0)),
            pl.BlockSpec((_EIN, _HW), lambda i: (0, 0)),
            pl.BlockSpec((_HW, _DE), lambda i: (0, 0)),
            pl.BlockSpec((_DE, _DE), lambda i: (0, 0)),
            pl.BlockSpec((1, _EIN), lambda i: (0, 0)),
            pl.BlockSpec((1, _EIN), lambda i: (0, 0)),
            pl.BlockSpec((1, _HW), lambda i: (0, 0)),
            pl.BlockSpec((1, _DE), lambda i: (0, 0)),
        ],
        out_specs=[
            pl.BlockSpec((_BE, _DE), lambda i: (i, 0)),
            pl.BlockSpec((_BE, _SW), lambda i: (i, 0)),
        ],
        out_shape=[
            jax.ShapeDtypeStruct((nchunks * _C, _DE), jnp.float32),
            jax.ShapeDtypeStruct((nchunks * _C, _SW), jnp.float32),
        ],
    )(g, rs2d.reshape(nblk, _CPB, _C), ss2d.reshape(nblk, _CPB, _C),
      e, eW1, eW2, edge_linear,
      e_ln_w.reshape(1, _EIN), e_ln_b.reshape(1, _EIN),
      eb1.reshape(1, _HW), eb2.reshape(1, _DE))


def _scatter_body(off, nchunks, p_hbm, col_hbm, init_hbm, acc_hbm,
                  idx_v, vbuf, shared):
    cid = lax.axis_index("c")
    sid = lax.axis_index("s")
    wid = sid * 2 + cid
    rbase = sid * _ROWS_PER_SUB
    pltpu.sync_copy(init_hbm.at[cid, pl.ds(rbase, _ROWS_PER_SUB)],
                    shared.at[pl.ds(rbase, _ROWS_PER_SUB)])
    plsc.subcore_barrier()

    q, r = divmod(nchunks, _NW)
    nloop = q + jnp.where(wid < r, 1, 0)
    s0 = wid * q + jnp.minimum(wid, r)

    def body(i, _):
        ch = s0 + i
        pltpu.sync_copy(col_hbm.at[pl.ds((off + ch) * _C, _C)], idx_v)
        pltpu.sync_copy(p_hbm.at[pl.ds(ch * _C, _C)], vbuf)
        pltpu.sync_copy(vbuf, shared.at[idx_v], add=True)
        return 0

    lax.fori_loop(0, nloop, body, 0)
    plsc.subcore_barrier()
    pltpu.sync_copy(shared.at[pl.ds(rbase, _ROWS_PER_SUB)],
                    acc_hbm.at[cid, pl.ds(rbase, _ROWS_PER_SUB)])


def _scatter(payload, col, init_hbm, off, nchunks):
    mesh = plsc.VectorSubcoreMesh(core_axis_name="c", subcore_axis_name="s")
    k = functools.partial(
        pl.kernel,
        mesh=mesh,
        out_type=jax.ShapeDtypeStruct((2, _NPAD, _SW), jnp.float32),
        scratch_types=[
            pltpu.VMEM((_C,), jnp.int32),
            pltpu.VMEM((_C, _SW), jnp.float32),
            pltpu.VMEM_SHARED((_NPAD, _SW), jnp.float32),
        ],
        compiler_params=pltpu.CompilerParams(use_tc_tiling_on_sc=False),
    )(functools.partial(_scatter_body, off, nchunks))
    return k(payload, col, init_hbm)


def _node_body(p0_ref, p1_ref, v_ref, w1_ref, w2_ref, nl_ref,
               lnw_ref, lnb_ref, b1_ref, b2_ref, vo_ref):
    p = p0_ref[0] + p1_ref[0]
    s = p[:, :_DE]
    cnt = p[:, _DE:_DE + 1]
    aggr = s / jnp.maximum(cnt, 1.0)
    vb = v_ref[...]
    m = (jnp.sum(aggr, axis=1, keepdims=True) + jnp.sum(vb, axis=1, keepdims=True)) / float(_NIN)
    da = aggr - m
    dv = vb - m
    var = (jnp.sum(da * da, axis=1, keepdims=True) + jnp.sum(dv * dv, axis=1, keepdims=True)) / float(_NIN)
    inv = lax.rsqrt(var + 1e-5)
    w = lnw_ref[0, :]
    bvec = lnb_ref[0, :]
    ga = da * inv * w[:_DE][None, :] + bvec[:_DE][None, :]
    gv = dv * inv * w[_DE:][None, :] + bvec[_DE:][None, :]
    pre = (jnp.dot(ga, w1_ref[:_DE, :], preferred_element_type=jnp.float32)
           + jnp.dot(gv, w1_ref[_DE:, :], preferred_element_type=jnp.float32)
           + b1_ref[0, :][None, :])
    h = jnp.maximum(pre, 0.0)
    h2 = jnp.maximum(
        jnp.dot(h, w2_ref[...], preferred_element_type=jnp.float32) + b2_ref[0, :][None, :],
        0.0)
    vo_ref[...] = jnp.dot(vb, nl_ref[...], preferred_element_type=jnp.float32) + h2


def _node_mlp(acc, v, nW1, nW2, node_linear, n_ln_w, n_ln_b, nb1, nb2):
    return pl.pallas_call(
        _node_body,
        grid=(_N // _BNO,),
        in_specs=[
            pl.BlockSpec((1, _BNO, _SW), lambda i: (0, i, 0)),
            pl.BlockSpec((1, _BNO, _SW), lambda i: (1, i, 0)),
            pl.BlockSpec((_BNO, _DN), lambda i: (i, 0)),
            pl.BlockSpec((_NIN, _HW), lambda i: (0, 0)),
            pl.BlockSpec((_HW, _DN), lambda i: (0, 0)),
            pl.BlockSpec((_DN, _DN), lambda i: (0, 0)),
            pl.BlockSpec((1, _NIN), lambda i: (0, 0)),
            pl.BlockSpec((1, _NIN), lambda i: (0, 0)),
            pl.BlockSpec((1, _HW), lambda i: (0, 0)),
            pl.BlockSpec((1, _DN), lambda i: (0, 0)),
        ],
        out_specs=pl.BlockSpec((_BNO, _DN), lambda i: (i, 0)),
        out_shape=jax.ShapeDtypeStruct((_N, _DN), jnp.float32),
    )(acc, acc, v, nW1, nW2, node_linear,
      n_ln_w.reshape(1, _NIN), n_ln_b.reshape(1, _NIN),
      nb1.reshape(1, _HW), nb2.reshape(1, _DN))


def kernel(v, e, edge_index, e_ln_w, e_ln_b, eW1, eb1, eW2, eb2, edge_linear,
           n_ln_w, n_ln_b, nW1, nb1, nW2, nb2, node_linear):
    row = edge_index[0].astype(jnp.int32)
    col = edge_index[1].astype(jnp.int32)
    vpad = jnp.pad(v, ((0, _NP - _N), (0, 0)))
    tbl, utbl, stbl = _make_tables(vpad, eW1, e_ln_w)

    acc = jnp.zeros((2, _NPAD, _SW), jnp.float32)
    eos = []
    splits = [(0, 1200), (1200, 800), (2000, 500)]
    for off, nc in splits:
        g, rs2d, ss2d = _gather(tbl, utbl, stbl, row, col, off, nc)
        eo, pay = _edge_mlp(g, rs2d, ss2d, e, eW1, eW2, edge_linear,
                            e_ln_w, e_ln_b, eb1, eb2, off, nc)
        eos.append(eo)
        acc = _scatter(pay, col, acc, off, nc)

    e_out = jnp.concatenate(eos, axis=0)
    v_out = _node_mlp(acc, v, nW1, nW2, node_linear, n_ln_w, n_ln_b, nb1, nb2)
    return (v_out, e_out)


# speedup vs baseline: 1.3446x; 1.3446x over previous
import functools
import jax
import jax.numpy as jnp
from jax import lax
from jax.experimental import pallas as pl
from jax.experimental.pallas import tpu as pltpu, tpu_sc as plsc

_N = 10000
_NP = 10240
_E = 320000
_EH = _E // 2
_DN = 128
_DE = 16
_HW = 128
_EIN = _DE + 2 * _DN
_NIN = _DE + _DN
_STW = 16
_SW = 32

_C = 128
_NCHUNK = _E // _C
_HCHUNK = _EH // _C
_NW = 32
_NSUB = 16
_NPAD = 10240
_ROWS_PER_SUB = _NPAD // _NSUB

_BN = 2048
_BNO = 2000
_BE = 3200
_CPB = _BE // _C


def _table_body(v_ref, w1_ref, lnw_ref, t_ref, u_ref, s_ref):
    w = lnw_ref[0, :]
    vb = v_ref[...]
    rs = jnp.sum(vb, axis=1, keepdims=True)
    ss = jnp.sum(vb * vb, axis=1, keepdims=True)
    vr = vb * w[_DE:_DE + _DN][None, :]
    vc = vb * w[_DE + _DN:][None, :]
    t_ref[...] = jnp.dot(vr, w1_ref[_DE:_DE + _DN, :], preferred_element_type=jnp.float32)
    u_ref[...] = jnp.dot(vc, w1_ref[_DE + _DN:, :], preferred_element_type=jnp.float32)
    pad = jnp.zeros((vb.shape[0], _STW - 2), jnp.float32)
    s_ref[...] = jnp.concatenate([rs, ss, pad], axis=1)


def _make_tables(vpad, eW1, e_ln_w):
    return pl.pallas_call(
        _table_body,
        grid=(_NP // _BN,),
        in_specs=[
            pl.BlockSpec((_BN, _DN), lambda i: (i, 0)),
            pl.BlockSpec((_EIN, _HW), lambda i: (0, 0)),
            pl.BlockSpec((1, _EIN), lambda i: (0, 0)),
        ],
        out_specs=[
            pl.BlockSpec((_BN, _DN), lambda i: (i, 0)),
            pl.BlockSpec((_BN, _DN), lambda i: (i, 0)),
            pl.BlockSpec((_BN, _STW), lambda i: (i, 0)),
        ],
        out_shape=[
            jax.ShapeDtypeStruct((_NP, _DN), jnp.float32),
            jax.ShapeDtypeStruct((_NP, _DN), jnp.float32),
            jax.ShapeDtypeStruct((_NP, _STW), jnp.float32),
        ],
    )(vpad, eW1, e_ln_w.reshape(1, _EIN))


def _gather_body(off, nchunks, t_hbm, u_hbm, s_hbm, row_hbm, col_hbm,
                 g_hbm, rs_hbm, ss_hbm,
                 idx_r, idx_c, buf, sbuf, rsb, ssb,
                 sem_i, sem_t0, sem_t1, sem_s0, sem_s1, sem_w):
    cid = lax.axis_index("c")
    sid = lax.axis_index("s")
    wid = sid * 2 + cid
    q, r = divmod(nchunks, _NW)
    nc = q + jnp.where(wid < r, 1, 0)
    s0 = wid * q + jnp.minimum(wid, r)
    npairs = nc // 2
    odd = nc - npairs * 2
    iota = lax.iota(jnp.int32, 16)
    zer = jnp.zeros((16,), jnp.int32)
    one = jnp.ones((16,), jnp.int32)

    def extract(half, _):
        def scal(j, _):
            rows = half * _C + j * 16 + iota
            rsb[half, pl.ds(j * 16, 16)] = plsc.load_gather(sbuf, [rows, zer])
            ssb[half, pl.ds(j * 16, 16)] = plsc.load_gather(sbuf, [rows, one])
            return 0
        lax.fori_loop(0, _C // 16, scal, 0)

    def pair(i, _):
        ch = s0 + i * 2
        base = (off + ch) * _C
        gbase = ch * _C

        pltpu.async_copy(row_hbm.at[pl.ds(base, 2 * _C)], idx_r, sem_i)
        pltpu.async_copy(col_hbm.at[pl.ds(base, 2 * _C)], idx_c, sem_i)

        @pl.when(i > 0)
        def _():
            pltpu.make_async_copy(buf, g_hbm.at[pl.ds(gbase, 2 * _C)], sem_w).wait()
            pltpu.make_async_copy(rsb, rs_hbm.at[pl.ds(ch, 2)], sem_w).wait()
            pltpu.make_async_copy(ssb, ss_hbm.at[pl.ds(ch, 2)], sem_w).wait()

        pltpu.make_async_copy(row_hbm.at[pl.ds(base, 2 * _C)], idx_r, sem_i).wait()
        pltpu.make_async_copy(col_hbm.at[pl.ds(base, 2 * _C)], idx_c, sem_i).wait()

        ir0 = idx_r.at[pl.ds(0, _C)]
        ir1 = idx_r.at[pl.ds(_C, _C)]
        ic0 = idx_c.at[pl.ds(0, _C)]
        ic1 = idx_c.at[pl.ds(_C, _C)]
        b0 = buf.at[pl.ds(0, _C)]
        b1 = buf.at[pl.ds(_C, _C)]
        sb0 = sbuf.at[pl.ds(0, _C)]
        sb1 = sbuf.at[pl.ds(_C, _C)]

        d_t0 = pltpu.async_copy(t_hbm.at[ir0], b0, sem_t0)
        d_t1 = pltpu.async_copy(t_hbm.at[ir1], b1, sem_t1)
        d_s0 = pltpu.async_copy(s_hbm.at[ir0], sb0, sem_s0)
        d_s1 = pltpu.async_copy(s_hbm.at[ir1], sb1, sem_s1)
        d_t0.wait()
        d_u0 = pltpu.async_copy(u_hbm.at[ic0], b0, sem_t0, add=True)
        d_s0.wait()
        d_v0 = pltpu.async_copy(s_hbm.at[ic0], sb0, sem_s0, add=True)
        d_t1.wait()
        d_u1 = pltpu.async_copy(u_hbm.at[ic1], b1, sem_t1, add=True)
        d_s1.wait()
        d_v1 = pltpu.async_copy(s_hbm.at[ic1], sb1, sem_s1, add=True)
        d_u0.wait()
        d_v0.wait()
        extract(0, None)
        d_u1.wait()
        d_v1.wait()
        extract(1, None)
        pltpu.async_copy(buf, g_hbm.at[pl.ds(gbase, 2 * _C)], sem_w)
        pltpu.async_copy(rsb, rs_hbm.at[pl.ds(ch, 2)], sem_w)
        pltpu.async_copy(ssb, ss_hbm.at[pl.ds(ch, 2)], sem_w)
        return 0

    lax.fori_loop(0, npairs, pair, 0)

    @pl.when(odd == 1)
    def _():
        ch = s0 + npairs * 2
        base = (off + ch) * _C
        ir0 = idx_r.at[pl.ds(0, _C)]
        ic0 = idx_c.at[pl.ds(0, _C)]
        b0 = buf.at[pl.ds(0, _C)]
        sb0 = sbuf.at[pl.ds(0, _C)]
        pltpu.async_copy(row_hbm.at[pl.ds(base, _C)], ir0, sem_i)
        pltpu.async_copy(col_hbm.at[pl.ds(base, _C)], ic0, sem_i)
        pltpu.make_async_copy(buf, g_hbm.at[pl.ds(0, 2 * _C)], sem_w).wait()
        pltpu.make_async_copy(rsb, rs_hbm.at[pl.ds(0, 2)], sem_w).wait()
        pltpu.make_async_copy(ssb, ss_hbm.at[pl.ds(0, 2)], sem_w).wait()
        pltpu.make_async_copy(row_hbm.at[pl.ds(base, _C)], ir0, sem_i).wait()
        pltpu.make_async_copy(col_hbm.at[pl.ds(base, _C)], ic0, sem_i).wait()
        d_t0 = pltpu.async_copy(t_hbm.at[ir0], b0, sem_t0)
        d_s0 = pltpu.async_copy(s_hbm.at[ir0], sb0, sem_s0)
        d_t0.wait()
        d_u0 = pltpu.async_copy(u_hbm.at[ic0], b0, sem_t0, add=True)
        d_s0.wait()
        d_v0 = pltpu.async_copy(s_hbm.at[ic0], sb0, sem_s0, add=True)
        d_u0.wait()
        d_v0.wait()
        extract(0, None)
        pltpu.sync_copy(b0, g_hbm.at[pl.ds(ch * _C, _C)])
        pltpu.sync_copy(rsb.at[0], rs_hbm.at[ch])
        pltpu.sync_copy(ssb.at[0], ss_hbm.at[ch])

    @pl.when(odd == 0)
    def _():
        pltpu.make_async_copy(buf, g_hbm.at[pl.ds(0, 2 * _C)], sem_w).wait()
        pltpu.make_async_copy(rsb, rs_hbm.at[pl.ds(0, 2)], sem_w).wait()
        pltpu.make_async_copy(ssb, ss_hbm.at[pl.ds(0, 2)], sem_w).wait()


def _gather(tbl, utbl, stbl, row, col, off, nchunks):
    mesh = plsc.VectorSubcoreMesh(core_axis_name="c", subcore_axis_name="s")
    k = functools.partial(
        pl.kernel,
        mesh=mesh,
        out_type=[
            jax.ShapeDtypeStruct((nchunks * _C, _DN), jnp.float32),
            jax.ShapeDtypeStruct((nchunks, _C), jnp.float32),
            jax.ShapeDtypeStruct((nchunks, _C), jnp.float32),
        ],
        scratch_types=[
            pltpu.VMEM((2 * _C,), jnp.int32),
            pltpu.VMEM((2 * _C,), jnp.int32),
            pltpu.VMEM((2 * _C, _DN), jnp.float32),
            pltpu.VMEM((2 * _C, _STW), jnp.float32),
            pltpu.VMEM((2, _C), jnp.float32),
            pltpu.VMEM((2, _C), jnp.float32),
            pltpu.SemaphoreType.DMA,
            pltpu.SemaphoreType.DMA,
            pltpu.SemaphoreType.DMA,
            pltpu.SemaphoreType.DMA,
            pltpu.SemaphoreType.DMA,
            pltpu.SemaphoreType.DMA,
        ],
        compiler_params=pltpu.CompilerParams(use_tc_tiling_on_sc=False,
                                             needs_layout_passes=False),
    )(functools.partial(_gather_body, off, nchunks))
    return k(tbl, utbl, stbl, row, col)


def _colbcast(mat):
    t = mat.T
    return jnp.concatenate(
        [jnp.broadcast_to(t[:, a:a + 1], (_C, _DN)) for a in range(_CPB)], axis=0)


def _edge_body(g_ref, rs_ref, ss_ref, e_ref, w1_ref, w2_ref, el_ref, lnw_ref,
               lnb_ref, b1_ref, b2_ref, eo_ref, sc_ref):
    w = lnw_ref[0, :]
    bvec = lnb_ref[0, :]
    u = jnp.dot(w, w1_ref[...], preferred_element_type=jnp.float32)
    c = jnp.dot(bvec, w1_ref[...], preferred_element_type=jnp.float32)
    eb = e_ref[...]
    rs_e = jnp.sum(eb, axis=1, keepdims=True)
    ss_e = jnp.sum(eb * eb, axis=1, keepdims=True)
    ew = eb * w[:_DE][None, :]
    ze = jnp.dot(ew, w1_ref[:_DE, :], preferred_element_type=jnp.float32)
    m = (rs_e + _colbcast(rs_ref[0])) / float(_EIN)
    q = (ss_e + _colbcast(ss_ref[0])) / float(_EIN)
    var = q - m * m
    inv = lax.rsqrt(var + 1e-5)
    z2 = g_ref[...] + ze - m * u[None, :]
    h1 = jnp.maximum(z2 * inv + (c + b1_ref[0, :])[None, :], 0.0)
    h2 = jnp.maximum(
        jnp.dot(h1, w2_ref[...], preferred_element_type=jnp.float32) + b2_ref[0, :][None, :],
        0.0)
    eo = jnp.dot(eb, el_ref[...], preferred_element_type=jnp.float32) + h2
    eo_ref[...] = eo
    n = eo.shape[0]
    sc_ref[...] = jnp.concatenate(
        [eo, jnp.ones((n, 1), jnp.float32), jnp.zeros((n, _SW - _DE - 1), jnp.float32)],
        axis=1)


def _edge_mlp(g, rs2d, ss2d, e, eW1, eW2, edge_linear, e_ln_w, e_ln_b, eb1, eb2,
              off_chunks, nchunks):
    off_b = off_chunks // _CPB
    nblk = nchunks // _CPB
    return pl.pallas_call(
        _edge_body,
        grid=(nblk,),
        in_specs=[
            pl.BlockSpec((_BE, _DN), lambda i: (i, 0)),
            pl.BlockSpec((1, _CPB, _C), lambda i: (i, 0, 0)),
            pl.BlockSpec((1, _CPB, _C), lambda i: (i, 0, 0)),
            pl.BlockSpec((_BE, _DE), lambda i: (i + off_b, 0)),
            pl.BlockSpec((_EIN, _HW), lambda i: (0, 0)),
            pl.BlockSpec((_HW, _DE), lambda i: (0, 0)),
            pl.BlockSpec((_DE, _DE), lambda i: (0, 0)),
            pl.BlockSpec((1, _EIN), lambda i: (0, 0)),
            pl.BlockSpec((1, _EIN), lambda i: (0, 0)),
            pl.BlockSpec((1, _HW), lambda i: (0, 0)),
            pl.BlockSpec((1, _DE), lambda i: (0, 0)),
        ],
        out_specs=[
            pl.BlockSpec((_BE, _DE), lambda i: (i, 0)),
            pl.BlockSpec((_BE, _SW), lambda i: (i, 0)),
        ],
        out_shape=[
            jax.ShapeDtypeStruct((nchunks * _C, _DE), jnp.float32),
            jax.ShapeDtypeStruct((nchunks * _C, _SW), jnp.float32),
        ],
    )(g, rs2d.reshape(nblk, _CPB, _C), ss2d.reshape(nblk, _CPB, _C),
      e, eW1, eW2, edge_linear,
      e_ln_w.reshape(1, _EIN), e_ln_b.reshape(1, _EIN),
      eb1.reshape(1, _HW), eb2.reshape(1, _DE))


def _scatter_body(off, nchunks, p_hbm, col_hbm, init_hbm, acc_hbm,
                  idx_v, vbuf, shared):
    cid = lax.axis_index("c")
    sid = lax.axis_index("s")
    wid = sid * 2 + cid
    rbase = sid * _ROWS_PER_SUB
    pltpu.sync_copy(init_hbm.at[cid, pl.ds(rbase, _ROWS_PER_SUB)],
                    shared.at[pl.ds(rbase, _ROWS_PER_SUB)])
    plsc.subcore_barrier()

    q, r = divmod(nchunks, _NW)
    nloop = q + jnp.where(wid < r, 1, 0)
    s0 = wid * q + jnp.minimum(wid, r)

    def body(i, _):
        ch = s0 + i
        pltpu.sync_copy(col_hbm.at[pl.ds((off + ch) * _C, _C)], idx_v)
        pltpu.sync_copy(p_hbm.at[pl.ds(ch * _C, _C)], vbuf)
        pltpu.sync_copy(vbuf, shared.at[idx_v], add=True)
        return 0

    lax.fori_loop(0, nloop, body, 0)
    plsc.subcore_barrier()
    pltpu.sync_copy(shared.at[pl.ds(rbase, _ROWS_PER_SUB)],
                    acc_hbm.at[cid, pl.ds(rbase, _ROWS_PER_SUB)])


def _scatter(payload, col, init_hbm, off, nchunks):
    mesh = plsc.VectorSubcoreMesh(core_axis_name="c", subcore_axis_name="s")
    k = functools.partial(
        pl.kernel,
        mesh=mesh,
        out_type=jax.ShapeDtypeStruct((2, _NPAD, _SW), jnp.float32),
        scratch_types=[
            pltpu.VMEM((_C,), jnp.int32),
            pltpu.VMEM((_C, _SW), jnp.float32),
            pltpu.VMEM_SHARED((_NPAD, _SW), jnp.float32),
        ],
        compiler_params=pltpu.CompilerParams(use_tc_tiling_on_sc=False),
    )(functools.partial(_scatter_body, off, nchunks))
    return k(payload, col, init_hbm)


def _node_body(p0_ref, p1_ref, v_ref, w1_ref, w2_ref, nl_ref,
               lnw_ref, lnb_ref, b1_ref, b2_ref, vo_ref):
    p = p0_ref[0] + p1_ref[0]
    s = p[:, :_DE]
    cnt = p[:, _DE:_DE + 1]
    aggr = s / jnp.maximum(cnt, 1.0)
    vb = v_ref[...]
    m = (jnp.sum(aggr, axis=1, keepdims=True) + jnp.sum(vb, axis=1, keepdims=True)) / float(_NIN)
    da = aggr - m
    dv = vb - m
    var = (jnp.sum(da * da, axis=1, keepdims=True) + jnp.sum(dv * dv, axis=1, keepdims=True)) / float(_NIN)
    inv = lax.rsqrt(var + 1e-5)
    w = lnw_ref[0, :]
    bvec = lnb_ref[0, :]
    ga = da * inv * w[:_DE][None, :] + bvec[:_DE][None, :]
    gv = dv * inv * w[_DE:][None, :] + bvec[_DE:][None, :]
    pre = (jnp.dot(ga, w1_ref[:_DE, :], preferred_element_type=jnp.float32)
           + jnp.dot(gv, w1_ref[_DE:, :], preferred_element_type=jnp.float32)
           + b1_ref[0, :][None, :])
    h = jnp.maximum(pre, 0.0)
    h2 = jnp.maximum(
        jnp.dot(h, w2_ref[...], preferred_element_type=jnp.float32) + b2_ref[0, :][None, :],
        0.0)
    vo_ref[...] = jnp.dot(vb, nl_ref[...], preferred_element_type=jnp.float32) + h2


def _node_mlp(acc, v, nW1, nW2, node_linear, n_ln_w, n_ln_b, nb1, nb2):
    return pl.pallas_call(
        _node_body,
        grid=(_N // _BNO,),
        in_specs=[
            pl.BlockSpec((1, _BNO, _SW), lambda i: (0, i, 0)),
            pl.BlockSpec((1, _BNO, _SW), lambda i: (1, i, 0)),
            pl.BlockSpec((_BNO, _DN), lambda i: (i, 0)),
            pl.BlockSpec((_NIN, _HW), lambda i: (0, 0)),
            pl.BlockSpec((_HW, _DN), lambda i: (0, 0)),
            pl.BlockSpec((_DN, _DN), lambda i: (0, 0)),
            pl.BlockSpec((1, _NIN), lambda i: (0, 0)),
            pl.BlockSpec((1, _NIN), lambda i: (0, 0)),
            pl.BlockSpec((1, _HW), lambda i: (0, 0)),
            pl.BlockSpec((1, _DN), lambda i: (0, 0)),
        ],
        out_specs=pl.BlockSpec((_BNO, _DN), lambda i: (i, 0)),
        out_shape=jax.ShapeDtypeStruct((_N, _DN), jnp.float32),
    )(acc, acc, v, nW1, nW2, node_linear,
      n_ln_w.reshape(1, _NIN), n_ln_b.reshape(1, _NIN),
      nb1.reshape(1, _HW), nb2.reshape(1, _DN))


def kernel(v, e, edge_index, e_ln_w, e_ln_b, eW1, eb1, eW2, eb2, edge_linear,
           n_ln_w, n_ln_b, nW1, nb1, nW2, nb2, node_linear):
    row = edge_index[0].astype(jnp.int32)
    col = edge_index[1].astype(jnp.int32)
    vpad = jnp.pad(v, ((0, _NP - _N), (0, 0)))
    tbl, utbl, stbl = _make_tables(vpad, eW1, e_ln_w)

    acc = jnp.zeros((2, _NPAD, _SW), jnp.float32)
    eos = []
    splits = [(0, 1200), (1200, 800), (2000, 500)]
    for off, nc in splits:
        g, rs2d, ss2d = _gather(tbl, utbl, stbl, row, col, off, nc)
        eo, pay = _edge_mlp(g, rs2d, ss2d, e, eW1, eW2, edge_linear,
                            e_ln_w, e_ln_b, eb1, eb2, off, nc)
        eos.append(eo)
        acc = _scatter(pay, col, acc, off, nc)

    e_out = jnp.concatenate(eos, axis=0)
    v_out = _node_mlp(acc, v, nW1, nW2, node_linear, n_ln_w, n_ln_b, nb1, nb2)
    return (v_out, e_out)
